# Initial kernel scaffold; baseline (speedup 1.0000x reference)
import jax
import jax.numpy as jnp
from jax.experimental import pallas as pl


def kernel(x):
    raise NotImplementedError("write your pallas kernel here")


# speedup vs baseline: 1.9669x; 1.9669x over previous
import jax
import jax.numpy as jnp
from jax.experimental import pallas as pl
from jax.experimental.pallas import tpu as pltpu

_NUM_CLASSES = 10
_ROWS = 8192
_COLS = 2048
_BLK_ROWS = 512


def _rank_block_kernel(x_ref, s_ref, lut_ref, o_ref):
    x = x_ref[...]

    def c(j):
        return x > s_ref[j - 1:j, :]

    def L(v):
        return lut_ref[v:v + 1, :]

    hi = jnp.where(c(7),
                   jnp.where(c(9), L(9), jnp.where(c(8), L(8), L(7))),
                   jnp.where(c(6), L(6), L(5)))
    lo = jnp.where(c(2),
                   jnp.where(c(4), L(4), jnp.where(c(3), L(3), L(2))),
                   jnp.where(c(1), L(1), L(0)))
    o_ref[...] = jnp.where(c(5), hi, lo)


def kernel(x):
    num_classes = _NUM_CLASSES
    key = jax.random.key(42)
    k1, k2, k3, k4 = jax.random.split(key, 4)

    boundary_idx = jax.random.randint(k1, (num_classes - 1,), 0, x.shape[0])
    randomized = jax.random.uniform(k2, (x.shape[1],)) > 0.5
    perm = jax.random.permutation(k3, num_classes)
    reverse = jax.random.uniform(k4, (x.shape[1],)) > 0.5

    s = jnp.sort(x[boundary_idx], axis=0)
    lut = jnp.where(randomized[None, :], perm[:, None],
                    jnp.arange(num_classes, dtype=perm.dtype)[:, None])
    lut = jnp.where(reverse[None, :], num_classes - 1 - lut, lut)

    grid = _ROWS // _BLK_ROWS
    return pl.pallas_call(
        _rank_block_kernel,
        grid=(grid,),
        in_specs=[
            pl.BlockSpec((_BLK_ROWS, _COLS), lambda i: (i, 0)),
            pl.BlockSpec((num_classes - 1, _COLS), lambda i: (0, 0)),
            pl.BlockSpec((num_classes, _COLS), lambda i: (0, 0)),
        ],
        out_specs=pl.BlockSpec((_BLK_ROWS, _COLS), lambda i: (i, 0)),
        out_shape=jax.ShapeDtypeStruct((_ROWS, _COLS), jnp.int32),
    )(x, s, lut)


# speedup vs baseline: 2.1131x; 1.0743x over previous
import jax
import jax.numpy as jnp
from jax.experimental import pallas as pl
from jax.experimental.pallas import tpu as pltpu

_NUM_CLASSES = 10
_ROWS = 8192
_COLS = 2048
_BLK_COLS = 128


def _rank_block_kernel(x_ref, s_ref, lut_ref, o_ref):
    x = x_ref[...]

    def c(j):
        return x > s_ref[j - 1:j, :]

    def L(v):
        return lut_ref[v:v + 1, :]

    hi = jnp.where(c(7),
                   jnp.where(c(9), L(9), jnp.where(c(8), L(8), L(7))),
                   jnp.where(c(6), L(6), L(5)))
    lo = jnp.where(c(2),
                   jnp.where(c(4), L(4), jnp.where(c(3), L(3), L(2))),
                   jnp.where(c(1), L(1), L(0)))
    o_ref[...] = jnp.where(c(5), hi, lo)


def kernel(x):
    num_classes = _NUM_CLASSES
    key = jax.random.key(42)
    k1, k2, k3, k4 = jax.random.split(key, 4)

    boundary_idx = jax.random.randint(k1, (num_classes - 1,), 0, x.shape[0])
    randomized = jax.random.uniform(k2, (x.shape[1],)) > 0.5
    perm = jax.random.permutation(k3, num_classes)
    reverse = jax.random.uniform(k4, (x.shape[1],)) > 0.5

    s = jnp.sort(x[boundary_idx], axis=0)
    lut = jnp.where(randomized[None, :], perm[:, None],
                    jnp.arange(num_classes, dtype=perm.dtype)[:, None])
    lut = jnp.where(reverse[None, :], num_classes - 1 - lut, lut)

    grid = _COLS // _BLK_COLS
    return pl.pallas_call(
        _rank_block_kernel,
        grid=(grid,),
        in_specs=[
            pl.BlockSpec((_ROWS, _BLK_COLS), lambda j: (0, j)),
            pl.BlockSpec((num_classes - 1, _BLK_COLS), lambda j: (0, j)),
            pl.BlockSpec((num_classes, _BLK_COLS), lambda j: (0, j)),
        ],
        out_specs=pl.BlockSpec((_ROWS, _BLK_COLS), lambda j: (0, j)),
        out_shape=jax.ShapeDtypeStruct((_ROWS, _COLS), jnp.int32),
    )(x, s, lut)
